# Initial kernel scaffold; baseline (speedup 1.0000x reference)
import jax
import jax.numpy as jnp
from jax.experimental import pallas as pl


def kernel(x, pos, edge_index, msg_W1, msg_b1, msg_W2, msg_b2, pos_W1, pos_b1, pos_W2, pos_b2, upd_W1, upd_b1, upd_W2, upd_b2, ln_w, ln_b, coors_scale):
    raise NotImplementedError("write your pallas kernel here")


# speedup vs baseline: 3.8569x; 3.8569x over previous
import functools

import jax
import jax.numpy as jnp
from jax import lax
from jax.experimental import pallas as pl
from jax.experimental.pallas import tpu as pltpu
from jax.experimental.pallas import tpu_sc as plsc

_N = 10000
_E = 320000
_F = 128
_H = 64
_W = 80
_CW = 16
_NW = 32
_ET = _E // _NW
_CH = 80
_NCH = _ET // _CH
_NP = 10240
_NS = _NP // 16


def _mesh():
    return plsc.VectorSubcoreMesh(core_axis_name="c", subcore_axis_name="s")


_SC_PARAMS = pltpu.CompilerParams(use_tc_tiling_on_sc=False)


@functools.partial(
    pl.kernel,
    out_type=jax.ShapeDtypeStruct((4, _NP, _CW), jnp.float32),
    mesh=_mesh(),
    compiler_params=_SC_PARAMS,
    scratch_types=[
        pltpu.VMEM_SHARED((_NP, _CW), jnp.float32),
        pltpu.VMEM_SHARED((_NP, _CW), jnp.float32),
        pltpu.VMEM((_NCH, _CH), jnp.int32),
        pltpu.VMEM((_NCH, _CH), jnp.int32),
        pltpu.VMEM((_CH, _CW), jnp.float32),
        pltpu.VMEM((_NS, _CW), jnp.float32),
    ],
)
def _sc_counts(dst_hbm, src_hbm, out, acc_d, acc_s, idx_d, idx_s, ones_v, zb):
    cid = lax.axis_index("c")
    sid = lax.axis_index("s")
    wid = cid * 16 + sid
    one_row = jnp.where(lax.iota(jnp.int32, 16) == 0, 1.0, 0.0).astype(jnp.float32)
    zero_row = jnp.zeros((16,), jnp.float32)

    def fill(i, _):
        ones_v[i, :] = one_row
        return ()

    lax.fori_loop(0, _CH, fill, ())

    def zfill(i, _):
        zb[i, :] = zero_row
        return ()

    lax.fori_loop(0, _NS, zfill, ())
    pltpu.sync_copy(zb, acc_d.at[pl.ds(sid * _NS, _NS)])
    pltpu.sync_copy(zb, acc_s.at[pl.ds(sid * _NS, _NS)])
    plsc.subcore_barrier()
    pltpu.sync_copy(dst_hbm.at[wid], idx_d)
    pltpu.sync_copy(src_hbm.at[wid], idx_s)

    def body(j, _):
        pltpu.sync_copy(ones_v, acc_d.at[idx_d.at[j]], add=True)
        pltpu.sync_copy(ones_v, acc_s.at[idx_s.at[j]], add=True)
        return ()

    lax.fori_loop(0, _NCH, body, ())
    plsc.subcore_barrier()
    sl = pl.ds(sid * _NS, _NS)
    pltpu.sync_copy(acc_d.at[sl], out.at[2 * cid, sl])
    pltpu.sync_copy(acc_s.at[sl], out.at[2 * cid + 1, sl])


@functools.partial(
    pl.kernel,
    out_type=(
        jax.ShapeDtypeStruct((_E, _W), jnp.float32),
        jax.ShapeDtypeStruct((_E, _W), jnp.float32),
    ),
    mesh=_mesh(),
    compiler_params=_SC_PARAMS,
    scratch_types=[
        pltpu.VMEM((_NCH, _CH), jnp.int32),
        pltpu.VMEM((_NCH, _CH), jnp.int32),
        pltpu.VMEM((_CH, _W), jnp.float32),
        pltpu.VMEM((_CH, _W), jnp.float32),
        pltpu.SemaphoreType.DMA,
        pltpu.SemaphoreType.DMA,
    ],
)
def _sc_gather(ti_hbm, tj_hbm, dst_hbm, src_hbm, gi_out, gj_out,
               idx_d, idx_s, bi, bj, sem1, sem2):
    cid = lax.axis_index("c")
    sid = lax.axis_index("s")
    wid = cid * 16 + sid
    pltpu.sync_copy(dst_hbm.at[wid], idx_d)
    pltpu.sync_copy(src_hbm.at[wid], idx_s)

    def body(j, _):
        base = wid * _ET + j * _CH
        c1 = pltpu.async_copy(ti_hbm.at[idx_d.at[j]], bi, sem1)
        c2 = pltpu.async_copy(tj_hbm.at[idx_s.at[j]], bj, sem2)
        c1.wait()
        c2.wait()
        pltpu.sync_copy(bi, gi_out.at[pl.ds(base, _CH)])
        pltpu.sync_copy(bj, gj_out.at[pl.ds(base, _CH)])
        return ()

    lax.fori_loop(0, _NCH, body, ())


@functools.partial(
    pl.kernel,
    out_type=jax.ShapeDtypeStruct((2, _NP, _W), jnp.float32),
    mesh=_mesh(),
    compiler_params=_SC_PARAMS,
    scratch_types=[
        pltpu.VMEM_SHARED((_NP, _W), jnp.float32),
        pltpu.VMEM((_NCH, _CH), jnp.int32),
        pltpu.VMEM((_CH, _W), jnp.float32),
        pltpu.VMEM((_NS, _W), jnp.float32),
    ],
)
def _sc_scatter(mm_hbm, dst_hbm, out, acc, idx_d, buf, zb):
    cid = lax.axis_index("c")
    sid = lax.axis_index("s")
    wid = cid * 16 + sid
    zero_row = jnp.zeros((16,), jnp.float32)

    def zfill(i, _):
        for cc in range(_W // 16):
            zb[i, pl.ds(cc * 16, 16)] = zero_row
        return ()

    lax.fori_loop(0, _NS, zfill, ())
    sl = pl.ds(sid * _NS, _NS)
    pltpu.sync_copy(zb, acc.at[sl])
    plsc.subcore_barrier()
    pltpu.sync_copy(dst_hbm.at[wid], idx_d)

    def body(j, _):
        base = wid * _ET + j * _CH
        pltpu.sync_copy(mm_hbm.at[pl.ds(base, _CH)], buf)
        pltpu.sync_copy(buf, acc.at[idx_d.at[j]], add=True)
        return ()

    lax.fori_loop(0, _NCH, body, ())
    plsc.subcore_barrier()
    pltpu.sync_copy(acc.at[sl], out.at[cid, sl])


def _tc_prep(x, pos, cnt4, lw, lb, Ai, Aj, b1):
    ef = float(_E) * float(_F)

    def body(x_ref, pos_ref, cnt_ref, lw_ref, lb_ref, ai_ref, aj_ref, b1_ref,
             pi_ref, pj_ref, cd_ref):
        xv = x_ref[...]
        cnt = cnt_ref[...]
        cd = cnt[0, :_N, 0:1] + cnt[2, :_N, 0:1]
        cs = cnt[1, :_N, 0:1] + cnt[3, :_N, 0:1]
        s1 = jnp.sum(xv, axis=1, keepdims=True)
        s2 = jnp.sum(xv * xv, axis=1, keepdims=True)
        mu_i = jnp.sum(cd * s1) / ef
        ms_i = jnp.sum(cd * s2) / ef
        mu_j = jnp.sum(cs * s1) / ef
        ms_j = jnp.sum(cs * s2) / ef
        rs_i = lax.rsqrt(ms_i - mu_i * mu_i + 1e-5)
        rs_j = lax.rsqrt(ms_j - mu_j * mu_j + 1e-5)
        lwv = lw_ref[...]
        lbv = lb_ref[...]
        a_i = lwv * rs_i
        b_i = lbv - mu_i * a_i
        a_j = lwv * rs_j
        b_j = lbv - mu_j * a_j
        aiw = ai_ref[...]
        ajw = aj_ref[...]
        pi = jnp.dot(xv * a_i, aiw, preferred_element_type=jnp.float32)
        pj = jnp.dot(xv * a_j, ajw, preferred_element_type=jnp.float32)
        cvec = (jnp.dot(b_i, aiw, preferred_element_type=jnp.float32)
                + jnp.dot(b_j, ajw, preferred_element_type=jnp.float32)
                + b1_ref[...])
        posv = pos_ref[...]
        zpad = jnp.zeros((_N, _W - _H - 3), jnp.float32)
        pi_ref[...] = jnp.concatenate([pi + cvec, posv, zpad], axis=1)
        pj_ref[...] = jnp.concatenate([pj, -posv, zpad], axis=1)
        cd_ref[...] = cd

    return pl.pallas_call(
        body,
        out_shape=(
            jax.ShapeDtypeStruct((_N, _W), jnp.float32),
            jax.ShapeDtypeStruct((_N, _W), jnp.float32),
            jax.ShapeDtypeStruct((_N, 1), jnp.float32),
        ),
    )(x, pos, cnt4, lw, lb, Ai, Aj, b1)


def _tc_edge(gi, gj, v, W2, b2, P1, pb1, p2t, pb2, cs):
    blk = 2000
    grid = _E // blk

    def body(gi_ref, gj_ref, v_ref, w2_ref, b2_ref, p1_ref, pb1_ref,
             p2t_ref, pb2_ref, cs_ref, out_ref):
        s = gi_ref[...] + gj_ref[...]
        q = s[:, 0:_H]
        dirv = s[:, _H:_H + 3]
        ss = jnp.sum(dirv * dirv, axis=1, keepdims=True)
        d = jnp.sqrt(ss)
        h = jnp.maximum(q + d * v_ref[...], 0.0)
        m = jnp.dot(h, w2_ref[...], preferred_element_type=jnp.float32) + b2_ref[...]
        t = jnp.maximum(
            jnp.dot(m, p1_ref[...], preferred_element_type=jnp.float32) + pb1_ref[...],
            0.0)
        g = jnp.sum(t * p2t_ref[...], axis=1, keepdims=True) + pb2_ref[...]
        w_e = cs_ref[...] * g / jnp.maximum(d, 1e-8)
        pm = dirv * w_e
        zpad = jnp.zeros((blk, _W - _H - 3), jnp.float32)
        out_ref[...] = jnp.concatenate([m, pm, zpad], axis=1)

    full = lambda shape: pl.BlockSpec(shape, lambda i: (0, 0))
    return pl.pallas_call(
        body,
        grid=(grid,),
        in_specs=[
            pl.BlockSpec((blk, _W), lambda i: (i, 0)),
            pl.BlockSpec((blk, _W), lambda i: (i, 0)),
            full((1, _H)),
            full((_H, _H)),
            full((1, _H)),
            full((_H, _H)),
            full((1, _H)),
            full((1, _H)),
            full((1, 1)),
            full((1, 1)),
        ],
        out_specs=pl.BlockSpec((blk, _W), lambda i: (i, 0)),
        out_shape=jax.ShapeDtypeStruct((_E, _W), jnp.float32),
    )(gi, gj, v, W2, b2, P1, pb1, p2t, pb2, cs)


def _tc_update(x, pos, part, cd, U1a, U1b, ub1, U2, ub2):
    def body(x_ref, pos_ref, part_ref, cd_ref, u1a_ref, u1b_ref, ub1_ref,
             u2_ref, ub2_ref, x_out, pos_out):
        p = part_ref[...]
        psum = p[0, :_N] + p[1, :_N]
        aggm = psum[:, 0:_H]
        aggp = psum[:, _H:_H + 3]
        cdv = jnp.maximum(cd_ref[...], 1.0)
        xv = x_ref[...]
        t = jnp.maximum(
            jnp.dot(xv, u1a_ref[...], preferred_element_type=jnp.float32)
            + jnp.dot(aggm, u1b_ref[...], preferred_element_type=jnp.float32)
            + ub1_ref[...], 0.0)
        x_out[...] = (jnp.dot(t, u2_ref[...], preferred_element_type=jnp.float32)
                      + ub2_ref[...] + xv)
        pos_out[...] = pos_ref[...] + aggp / cdv

    return pl.pallas_call(
        body,
        out_shape=(
            jax.ShapeDtypeStruct((_N, _F), jnp.float32),
            jax.ShapeDtypeStruct((_N, 3), jnp.float32),
        ),
    )(x, pos, part, cd, U1a, U1b, ub1, U2, ub2)


def kernel(x, pos, edge_index, msg_W1, msg_b1, msg_W2, msg_b2,
           pos_W1, pos_b1, pos_W2, pos_b2, upd_W1, upd_b1, upd_W2, upd_b2,
           ln_w, ln_b, coors_scale):
    src_r = edge_index[0].reshape(_NW, _NCH, _CH)
    dst_r = edge_index[1].reshape(_NW, _NCH, _CH)
    cnt4 = _sc_counts(dst_r, src_r)
    for l in range(2):
        Ai = msg_W1[l, :_F]
        Aj = msg_W1[l, _F:2 * _F]
        v = msg_W1[l, 2 * _F:2 * _F + 1]
        pi_cat, pj_cat, cd = _tc_prep(
            x, pos, cnt4, ln_w[l][None], ln_b[l][None], Ai, Aj, msg_b1[l][None])
        gi, gj = _sc_gather(pi_cat, pj_cat, dst_r, src_r)
        mm = _tc_edge(gi, gj, v, msg_W2[l], msg_b2[l][None],
                      pos_W1[l], pos_b1[l][None], pos_W2[l].T, pos_b2[l][None],
                      coors_scale[l].reshape(1, 1))
        part = _sc_scatter(mm, dst_r)
        x, pos = _tc_update(x, pos, part, cd,
                            upd_W1[l, :_F], upd_W1[l, _F:], upd_b1[l][None],
                            upd_W2[l], upd_b2[l][None])
    return (x, pos)
